# Initial kernel scaffold; baseline (speedup 1.0000x reference)
import jax
import jax.numpy as jnp
from jax.experimental import pallas as pl


def kernel(bucket_goods_raw, realtime_back_category, realtime_goods, realtime_pair_click, realtime_passtime, realtime_user_group, pair_feature, bucket_user_cspu_obj, bucket_goods_box_obj, bucket_pair_box_obj, bucket_ozid_cspu_obj, bucket_user_behavior_obj, bucket_goods_gross_obj, W_raw, b_raw, T_pair, T_back, T_goods, T_click, T_pairbox, T_goodsbox, T_passtime, T_ucspu, T_ozid, T_ubeh, T_gross, T_ugroup):
    raise NotImplementedError("write your pallas kernel here")


# speedup vs baseline: 4.6019x; 4.6019x over previous
import jax
import jax.numpy as jnp
from jax import lax
from jax.experimental import pallas as pl
from jax.experimental.pallas import tpu as pltpu
from jax.experimental.pallas import tpu_sc as plsc

B = 4096
NC, NS, L = 2, 16, 16
NW = NC * NS
BPW = B // NW
NG = BPW // L
D_RAW = 64

FEAT_F = (20, 10, 30, 1, 15, 15, 8, 20, 20, 26, 12, 6)
N_FEAT = len(FEAT_F)
N_COL = 13


def _sc_kernel_body(*refs):
    raw_hbm, wb_hbm = refs[0], refs[1]
    idx_hbm = refs[2:2 + N_FEAT]
    tab_hbm = refs[2 + N_FEAT:2 + 2 * N_FEAT]
    out_hbm = refs[2 + 2 * N_FEAT]
    s = refs[3 + 2 * N_FEAT:]
    raw_v, wb_v = s[0], s[1]
    idx_v = s[2:2 + N_FEAT]
    val_v = s[2 + N_FEAT:2 + 2 * N_FEAT]
    out_v = s[2 + 2 * N_FEAT]
    sem_in = s[3 + 2 * N_FEAT]
    sem_g = s[4 + 2 * N_FEAT]

    wid = lax.axis_index("s") * NC + lax.axis_index("c")
    base = wid * BPW

    pend = []
    pend.append(pltpu.async_copy(raw_hbm.at[:, pl.ds(base, BPW)], raw_v, sem_in))
    pend.append(pltpu.async_copy(wb_hbm, wb_v, sem_in))
    for i in range(N_FEAT):
        pend.append(pltpu.async_copy(idx_hbm[i].at[wid], idx_v[i], sem_in))
    for p in pend:
        p.wait()

    pend = []
    for i in range(N_FEAT):
        pend.append(pltpu.async_copy(tab_hbm[i].at[idx_v[i]], val_v[i], sem_g))
    for p in pend:
        p.wait()

    wvecs = [wb_v[pl.ds(k * L, L)] for k in range(D_RAW // L + 1)]
    bias = wvecs[D_RAW // L][0]
    accs = [jnp.full((L,), 0.0, jnp.float32) + bias for _ in range(NG)]
    for d in range(D_RAW):
        w = wvecs[d // L][d % L]
        for g in range(NG):
            accs[g] = accs[g] + raw_v[d, pl.ds(g * L, L)] * w
    for g in range(NG):
        out_v[0, pl.ds(g * L, L)] = accs[g]

    for i in range(N_FEAT):
        F = FEAT_F[i]
        for g in range(NG):
            acc = val_v[i][pl.ds(g * L, L)]
            for f in range(1, F):
                acc = acc + val_v[i][pl.ds(f * BPW + g * L, L)]
            out_v[1 + i, pl.ds(g * L, L)] = acc

    pltpu.sync_copy(out_v, out_hbm.at[:, pl.ds(base, BPW)])


@jax.jit
def _build_linear_sc(raw_t, wb, idx_ps, tables):
    scratch = [
        pltpu.VMEM((D_RAW, BPW), jnp.float32),
        pltpu.VMEM((D_RAW + 16,), jnp.float32),
    ]
    scratch += [pltpu.VMEM((f * BPW,), jnp.int32) for f in FEAT_F]
    scratch += [pltpu.VMEM((f * BPW,), jnp.float32) for f in FEAT_F]
    scratch += [
        pltpu.VMEM((N_COL, BPW), jnp.float32),
        pltpu.SemaphoreType.DMA,
        pltpu.SemaphoreType.DMA,
    ]
    mesh = plsc.VectorSubcoreMesh(core_axis_name="c", subcore_axis_name="s")
    run = pl.kernel(
        _sc_kernel_body,
        out_type=jax.ShapeDtypeStruct((N_COL, B), jnp.float32),
        mesh=mesh,
        scratch_types=scratch,
    )
    return run(raw_t, wb, *idx_ps, *tables)


def kernel(bucket_goods_raw, realtime_back_category, realtime_goods,
           realtime_pair_click, realtime_passtime, realtime_user_group,
           pair_feature, bucket_user_cspu_obj, bucket_goods_box_obj,
           bucket_pair_box_obj, bucket_ozid_cspu_obj, bucket_user_behavior_obj,
           bucket_goods_gross_obj, W_raw, b_raw, T_pair, T_back, T_goods,
           T_click, T_pairbox, T_goodsbox, T_passtime, T_ucspu, T_ozid,
           T_ubeh, T_gross, T_ugroup):
    feats = [
        (pair_feature, T_pair), (realtime_back_category, T_back),
        (realtime_goods, T_goods), (realtime_pair_click, T_click),
        (bucket_pair_box_obj, T_pairbox), (bucket_goods_box_obj, T_goodsbox),
        (realtime_passtime, T_passtime), (bucket_user_cspu_obj, T_ucspu),
        (bucket_ozid_cspu_obj, T_ozid), (bucket_user_behavior_obj, T_ubeh),
        (bucket_goods_gross_obj, T_gross), (realtime_user_group, T_ugroup),
    ]
    idx_ps = [
        idx.reshape(NW, BPW, -1).astype(jnp.int32)
        .transpose(0, 2, 1).reshape(NW, -1)
        for idx, _ in feats
    ]
    tables = [tab.reshape(-1).astype(jnp.float32) for _, tab in feats]
    raw_t = bucket_goods_raw.T
    wb = jnp.concatenate([
        W_raw.reshape(-1).astype(jnp.float32),
        b_raw.reshape(-1).astype(jnp.float32),
        jnp.zeros((15,), jnp.float32),
    ])
    out_t = _build_linear_sc(raw_t, wb, idx_ps, tables)
    return out_t.T
